# baseline (speedup 1.0000x reference)
import functools

import jax
import jax.numpy as jnp
from jax import lax
from jax.experimental import pallas as pl
from jax.experimental.pallas import tpu as pltpu
from jax.experimental.pallas import tpu_sc as plsc

_EMBED = 300
_VOCAB = 100000
_BATCH = 1024
_MAX_NORM = float(_EMBED)

_NBLK = 4096


def _gather_call(emb_table, idx):
    info = plsc.get_sparse_core_info()
    nc = info.num_cores
    b_per_c = _BATCH // nc
    mesh = plsc.ScalarSubcoreMesh(axis_name="c", num_cores=nc)

    @functools.partial(
        pl.kernel,
        mesh=mesh,
        out_type=jax.ShapeDtypeStruct((_BATCH, _EMBED), jnp.float32),
        scratch_types=[
            pltpu.SMEM((b_per_c,), jnp.int32),
            pltpu.SemaphoreType.DMA,
        ],
    )
    def gather_k(table_hbm, idx_hbm, out_hbm, idx_s, sem):
        base = lax.axis_index("c") * b_per_c
        pltpu.sync_copy(idx_hbm.at[pl.ds(base, b_per_c)], idx_s)

        def issue(i, _):
            pltpu.make_async_copy(
                table_hbm.at[pl.ds(idx_s[i], 1), :],
                out_hbm.at[pl.ds(base + i, 1), :],
                sem,
            ).start()
            return 0

        lax.fori_loop(0, b_per_c, issue, 0, unroll=8)

        pltpu.make_async_copy(
            table_hbm.at[pl.ds(0, b_per_c), :],
            out_hbm.at[pl.ds(base, b_per_c), :],
            sem,
        ).wait()

    return gather_k(emb_table, idx)


def _mm_body(x_ref, w_ref, b_ref, out_ref, xs_ref, _unused_ref):
    @pl.when(pl.program_id(0) == 0)
    def _():
        xv = x_ref[...]
        ss = jnp.sum(xv * xv, axis=1, keepdims=True)
        norm = jnp.sqrt(ss)
        scale = jnp.minimum(1.0, _MAX_NORM / jnp.maximum(norm, 1e-7))
        xs_ref[...] = (xv * scale).astype(jnp.float32)

    out_ref[...] = lax.dot_general(
        xs_ref[...],
        w_ref[...],
        dimension_numbers=(((1,), (1,)), ((), ())),
        preferred_element_type=jnp.float32,
    ) + b_ref[...][None, :]


def _matmul_call(x, W, b):
    nblocks = pl.cdiv(_VOCAB, _NBLK)
    return pl.pallas_call(
        _mm_body,
        grid=(nblocks,),
        in_specs=[
            pl.BlockSpec((_BATCH, _EMBED), lambda j: (0, 0)),
            pl.BlockSpec((_NBLK, _EMBED), lambda j: (j, 0)),
            pl.BlockSpec((_NBLK,), lambda j: (j,)),
        ],
        out_specs=pl.BlockSpec((_BATCH, _NBLK), lambda j: (0, j)),
        out_shape=jax.ShapeDtypeStruct((_BATCH, _VOCAB), jnp.float32),
        scratch_shapes=[
            pltpu.VMEM((_BATCH, _EMBED), jnp.float32),
            pltpu.VMEM((8, 128), jnp.float32),
        ],
    )(x, W, b)


def kernel(inputs, emb_table, W, b):
    x = _gather_call(emb_table, inputs.astype(jnp.int32))
    return _matmul_call(x, W, b)


# speedup vs baseline: 2.2332x; 2.2332x over previous
import functools

import jax
import jax.numpy as jnp
from jax import lax
from jax.experimental import pallas as pl
from jax.experimental.pallas import tpu as pltpu
from jax.experimental.pallas import tpu_sc as plsc

_EMBED = 300
_VOCAB = 100000
_BATCH = 1024
_MAX_NORM = float(_EMBED)

_NBLK = 4096


def _gather_call(emb_table, idx):
    info = plsc.get_sparse_core_info()
    nc = info.num_cores
    b_per_c = _BATCH // nc
    mesh = plsc.ScalarSubcoreMesh(axis_name="c", num_cores=nc)

    @functools.partial(
        pl.kernel,
        mesh=mesh,
        out_type=jax.ShapeDtypeStruct((_BATCH, _EMBED), jnp.float32),
        scratch_types=[
            pltpu.SMEM((b_per_c,), jnp.int32),
            pltpu.SemaphoreType.DMA,
        ],
    )
    def gather_k(table_hbm, idx_hbm, out_hbm, idx_s, sem):
        base = lax.axis_index("c") * b_per_c
        pltpu.sync_copy(idx_hbm.at[pl.ds(base, b_per_c)], idx_s)

        def issue(i, _):
            pltpu.make_async_copy(
                table_hbm.at[pl.ds(idx_s[i], 1), :],
                out_hbm.at[pl.ds(base + i, 1), :],
                sem,
            ).start()
            return 0

        lax.fori_loop(0, b_per_c, issue, 0, unroll=8)

        pltpu.make_async_copy(
            table_hbm.at[pl.ds(0, b_per_c), :],
            out_hbm.at[pl.ds(base, b_per_c), :],
            sem,
        ).wait()

    return gather_k(emb_table, idx)


def _mm_body(x_ref, w_ref, b_ref, out_ref, xs_ref):
    @pl.when(pl.program_id(0) == 0)
    def _():
        xv = x_ref[...]
        ss = jnp.sum(xv * xv, axis=1, keepdims=True)
        norm = jnp.sqrt(ss)
        scale = jnp.minimum(1.0, _MAX_NORM / jnp.maximum(norm, 1e-7))
        xs_ref[...] = jnp.transpose(xv * scale)

    out_ref[...] = lax.dot_general(
        w_ref[...],
        xs_ref[...],
        dimension_numbers=(((0,), (0,)), ((), ())),
        preferred_element_type=jnp.float32,
    ) + b_ref[...][:, None]


def _matmul_call(x, W, b):
    wt = jnp.transpose(W)
    nblocks = pl.cdiv(_VOCAB, _NBLK)
    out_t = pl.pallas_call(
        _mm_body,
        grid=(nblocks,),
        in_specs=[
            pl.BlockSpec((_BATCH, _EMBED), lambda j: (0, 0)),
            pl.BlockSpec((_EMBED, _NBLK), lambda j: (0, j)),
            pl.BlockSpec((_NBLK,), lambda j: (j,)),
        ],
        out_specs=pl.BlockSpec((_NBLK, _BATCH), lambda j: (j, 0)),
        out_shape=jax.ShapeDtypeStruct((_VOCAB, _BATCH), jnp.float32),
        scratch_shapes=[
            pltpu.VMEM((_EMBED, _BATCH), jnp.float32),
        ],
    )(x, wt, b)
    return jnp.transpose(out_t)


def kernel(inputs, emb_table, W, b):
    x = _gather_call(emb_table, inputs.astype(jnp.int32))
    return _matmul_call(x, W, b)


# speedup vs baseline: 2.4337x; 1.0898x over previous
import functools

import jax
import jax.numpy as jnp
from jax import lax
from jax.experimental import pallas as pl
from jax.experimental.pallas import tpu as pltpu
from jax.experimental.pallas import tpu_sc as plsc

_EMBED = 300
_VOCAB = 100000
_BATCH = 1024
_MAX_NORM = float(_EMBED)

_NBLK = 4096


_TBLK = 2048


def _tr_body(tt_ref, o_ref):
    o_ref[...] = jnp.transpose(tt_ref[...])


def _transpose_call(table_t):
    nblocks = pl.cdiv(_VOCAB, _TBLK)
    return pl.pallas_call(
        _tr_body,
        grid=(nblocks,),
        in_specs=[pl.BlockSpec((_EMBED, _TBLK), lambda j: (0, j))],
        out_specs=pl.BlockSpec((_TBLK, _EMBED), lambda j: (j, 0)),
        out_shape=jax.ShapeDtypeStruct((_VOCAB, _EMBED), jnp.float32),
    )(table_t)


def _gather_call(emb_table, idx):
    info = plsc.get_sparse_core_info()
    nc = info.num_cores
    b_per_c = _BATCH // nc
    mesh = plsc.ScalarSubcoreMesh(axis_name="c", num_cores=nc)

    @functools.partial(
        pl.kernel,
        mesh=mesh,
        out_type=jax.ShapeDtypeStruct((_BATCH, _EMBED), jnp.float32),
        scratch_types=[
            pltpu.SMEM((b_per_c,), jnp.int32),
            pltpu.SemaphoreType.DMA,
        ],
    )
    def gather_k(table_hbm, idx_hbm, out_hbm, idx_s, sem):
        base = lax.axis_index("c") * b_per_c
        pltpu.sync_copy(idx_hbm.at[pl.ds(base, b_per_c)], idx_s)

        def issue(i, _):
            pltpu.make_async_copy(
                table_hbm.at[pl.ds(idx_s[i], 1), :],
                out_hbm.at[pl.ds(base + i, 1), :],
                sem,
            ).start()
            return 0

        lax.fori_loop(0, b_per_c, issue, 0, unroll=8)

        pltpu.make_async_copy(
            table_hbm.at[pl.ds(0, b_per_c), :],
            out_hbm.at[pl.ds(base, b_per_c), :],
            sem,
        ).wait()

    return gather_k(emb_table, idx)


def _mm_body(x_ref, w_ref, b_ref, out_ref, xs_ref):
    @pl.when(pl.program_id(0) == 0)
    def _():
        xv = x_ref[...]
        ss = jnp.sum(xv * xv, axis=1, keepdims=True)
        norm = jnp.sqrt(ss)
        scale = jnp.minimum(1.0, _MAX_NORM / jnp.maximum(norm, 1e-7))
        xs_ref[...] = jnp.transpose(xv * scale)

    out_ref[...] = lax.dot_general(
        w_ref[...],
        xs_ref[...],
        dimension_numbers=(((0,), (0,)), ((), ())),
        preferred_element_type=jnp.float32,
    ) + b_ref[...][:, None]


def _matmul_call(x, W, b):
    wt = jnp.transpose(W)
    nblocks = pl.cdiv(_VOCAB, _NBLK)
    out_t = pl.pallas_call(
        _mm_body,
        grid=(nblocks,),
        in_specs=[
            pl.BlockSpec((_BATCH, _EMBED), lambda j: (0, 0)),
            pl.BlockSpec((_EMBED, _NBLK), lambda j: (0, j)),
            pl.BlockSpec((_NBLK,), lambda j: (j,)),
        ],
        out_specs=pl.BlockSpec((_NBLK, _BATCH), lambda j: (j, 0)),
        out_shape=jax.ShapeDtypeStruct((_VOCAB, _BATCH), jnp.float32),
        scratch_shapes=[
            pltpu.VMEM((_EMBED, _BATCH), jnp.float32),
        ],
    )(x, wt, b)
    return jnp.transpose(out_t)


def kernel(inputs, emb_table, W, b):
    table_rm = _transpose_call(jnp.transpose(emb_table))
    x = _gather_call(table_rm, inputs.astype(jnp.int32))
    return _matmul_call(x, W, b)


# speedup vs baseline: 2.8259x; 1.1612x over previous
import functools

import jax
import jax.numpy as jnp
from jax import lax
from jax.experimental import pallas as pl
from jax.experimental.pallas import tpu as pltpu
from jax.experimental.pallas import tpu_sc as plsc

_EMBED = 300
_EMBED_PAD = 384
_VOCAB = 100000
_BATCH = 1024
_MAX_NORM = float(_EMBED)

_NBLK = 4096
_TBLK = 2048


def _tr_body(tt_ref, o_ref):
    o_ref[:, :_EMBED] = jnp.transpose(tt_ref[...])
    o_ref[:, _EMBED:] = jnp.zeros((_TBLK, _EMBED_PAD - _EMBED), jnp.float32)


def _transpose_call(table_t):
    nblocks = pl.cdiv(_VOCAB, _TBLK)
    return pl.pallas_call(
        _tr_body,
        grid=(nblocks,),
        in_specs=[pl.BlockSpec((_EMBED, _TBLK), lambda j: (0, j))],
        out_specs=pl.BlockSpec((_TBLK, _EMBED_PAD), lambda j: (j, 0)),
        out_shape=jax.ShapeDtypeStruct((_VOCAB, _EMBED_PAD), jnp.float32),
    )(table_t)


def _gather_call(table_pad, idx):
    info = plsc.get_sparse_core_info()
    nc, ns = info.num_cores, info.num_subcores
    nw = nc * ns
    b_per_w = _BATCH // nw
    mesh = plsc.VectorSubcoreMesh(core_axis_name="c", subcore_axis_name="s")

    @functools.partial(
        pl.kernel,
        mesh=mesh,
        out_type=jax.ShapeDtypeStruct((_BATCH, _EMBED_PAD), jnp.float32),
        scratch_types=[
            pltpu.VMEM((b_per_w,), jnp.int32),
            pltpu.VMEM((b_per_w, _EMBED_PAD), jnp.float32),
            pltpu.SemaphoreType.DMA,
        ],
    )
    def gather_k(table_hbm, idx_hbm, out_hbm, idx_v, rows_v, sem):
        wid = lax.axis_index("s") * nc + lax.axis_index("c")
        base = wid * b_per_w
        pltpu.sync_copy(idx_hbm.at[pl.ds(base, b_per_w)], idx_v)
        pltpu.async_copy(table_hbm.at[idx_v], rows_v, sem).wait()
        pltpu.sync_copy(rows_v, out_hbm.at[pl.ds(base, b_per_w)])

    return gather_k(table_pad, idx)


def _mm_body(x_ref, w_ref, b_ref, out_ref, xs_ref):
    @pl.when(pl.program_id(0) == 0)
    def _():
        xv = x_ref[:, :_EMBED]
        ss = jnp.sum(xv * xv, axis=1, keepdims=True)
        norm = jnp.sqrt(ss)
        scale = jnp.minimum(1.0, _MAX_NORM / jnp.maximum(norm, 1e-7))
        xs_ref[...] = jnp.transpose(xv * scale)

    out_ref[...] = lax.dot_general(
        w_ref[...],
        xs_ref[...],
        dimension_numbers=(((0,), (0,)), ((), ())),
        preferred_element_type=jnp.float32,
    ) + b_ref[...][:, None]


def _matmul_call(x, W, b):
    wt = jnp.transpose(W)
    nblocks = pl.cdiv(_VOCAB, _NBLK)
    out_t = pl.pallas_call(
        _mm_body,
        grid=(nblocks,),
        in_specs=[
            pl.BlockSpec((_BATCH, _EMBED_PAD), lambda j: (0, 0)),
            pl.BlockSpec((_EMBED, _NBLK), lambda j: (0, j)),
            pl.BlockSpec((_NBLK,), lambda j: (j,)),
        ],
        out_specs=pl.BlockSpec((_NBLK, _BATCH), lambda j: (j, 0)),
        out_shape=jax.ShapeDtypeStruct((_VOCAB, _BATCH), jnp.float32),
        scratch_shapes=[
            pltpu.VMEM((_EMBED, _BATCH), jnp.float32),
        ],
    )(x, wt, b)
    return jnp.transpose(out_t)


def kernel(inputs, emb_table, W, b):
    table_pad = _transpose_call(jnp.transpose(emb_table))
    x = _gather_call(table_pad, inputs.astype(jnp.int32))
    return _matmul_call(x, W, b)


# speedup vs baseline: 2.9085x; 1.0292x over previous
import functools

import jax
import jax.numpy as jnp
from jax import lax
from jax.experimental import pallas as pl
from jax.experimental.pallas import tpu as pltpu
from jax.experimental.pallas import tpu_sc as plsc

_EMBED = 300
_EMBED_PAD = 384
_VOCAB = 100000
_BATCH = 1024
_MAX_NORM = float(_EMBED)

_NBLK = 4096
_TBLK = 4096


def _tr_body(tt_ref, o_ref):
    o_ref[:, :_EMBED] = jnp.transpose(tt_ref[...])
    o_ref[:, _EMBED:] = jnp.zeros((_TBLK, _EMBED_PAD - _EMBED), jnp.float32)


def _transpose_call(table_t):
    nblocks = pl.cdiv(_VOCAB, _TBLK)
    return pl.pallas_call(
        _tr_body,
        grid=(nblocks,),
        in_specs=[pl.BlockSpec((_EMBED, _TBLK), lambda j: (0, j))],
        out_specs=pl.BlockSpec((_TBLK, _EMBED_PAD), lambda j: (j, 0)),
        out_shape=jax.ShapeDtypeStruct((_VOCAB, _EMBED_PAD), jnp.float32),
    )(table_t)


def _gather_call(table_pad, idx):
    info = plsc.get_sparse_core_info()
    nc, ns = info.num_cores, info.num_subcores
    nw = nc * ns
    b_per_w = _BATCH // nw
    mesh = plsc.VectorSubcoreMesh(core_axis_name="c", subcore_axis_name="s")

    @functools.partial(
        pl.kernel,
        mesh=mesh,
        out_type=jax.ShapeDtypeStruct((_BATCH, _EMBED_PAD), jnp.float32),
        scratch_types=[
            pltpu.VMEM((b_per_w,), jnp.int32),
            pltpu.VMEM((b_per_w, _EMBED_PAD), jnp.float32),
            pltpu.SemaphoreType.DMA,
        ],
    )
    def gather_k(table_hbm, idx_hbm, out_hbm, idx_v, rows_v, sem):
        wid = lax.axis_index("s") * nc + lax.axis_index("c")
        base = wid * b_per_w
        pltpu.sync_copy(idx_hbm.at[pl.ds(base, b_per_w)], idx_v)
        pltpu.async_copy(table_hbm.at[idx_v], rows_v, sem).wait()
        pltpu.sync_copy(rows_v, out_hbm.at[pl.ds(base, b_per_w)])

    return gather_k(table_pad, idx)


def _mm_body(x_ref, w_ref, b_ref, out_ref, xs_ref):
    @pl.when(pl.program_id(0) == 0)
    def _():
        xv = x_ref[:, :_EMBED]
        ss = jnp.sum(xv * xv, axis=1, keepdims=True)
        norm = jnp.sqrt(ss)
        scale = jnp.minimum(1.0, _MAX_NORM / jnp.maximum(norm, 1e-7))
        xs_ref[...] = jnp.transpose(xv * scale)

    out_ref[...] = lax.dot_general(
        w_ref[...],
        xs_ref[...],
        dimension_numbers=(((0,), (0,)), ((), ())),
        preferred_element_type=jnp.float32,
    ) + b_ref[...][:, None]


def _matmul_call(x, W, b):
    wt = jnp.transpose(W)
    nblocks = pl.cdiv(_VOCAB, _NBLK)
    out_t = pl.pallas_call(
        _mm_body,
        grid=(nblocks,),
        in_specs=[
            pl.BlockSpec((_BATCH, _EMBED_PAD), lambda j: (0, 0)),
            pl.BlockSpec((_EMBED, _NBLK), lambda j: (0, j)),
            pl.BlockSpec((_NBLK,), lambda j: (j,)),
        ],
        out_specs=pl.BlockSpec((_NBLK, _BATCH), lambda j: (j, 0)),
        out_shape=jax.ShapeDtypeStruct((_VOCAB, _BATCH), jnp.float32),
        scratch_shapes=[
            pltpu.VMEM((_EMBED, _BATCH), jnp.float32),
        ],
    )(x, wt, b)
    return jnp.transpose(out_t)


def kernel(inputs, emb_table, W, b):
    table_pad = _transpose_call(jnp.transpose(emb_table))
    x = _gather_call(table_pad, inputs.astype(jnp.int32))
    return _matmul_call(x, W, b)


# speedup vs baseline: 2.9287x; 1.0069x over previous
import functools

import jax
import jax.numpy as jnp
from jax import lax
from jax.experimental import pallas as pl
from jax.experimental.pallas import tpu as pltpu
from jax.experimental.pallas import tpu_sc as plsc

_EMBED = 300
_EMBED_PAD = 384
_VOCAB = 100000
_BATCH = 1024
_MAX_NORM = float(_EMBED)

_NBLK = 4096
_TBLK = 8192


def _tr_body(tt_ref, o_ref):
    o_ref[:, :_EMBED] = jnp.transpose(tt_ref[...])
    o_ref[:, _EMBED:] = jnp.zeros((_TBLK, _EMBED_PAD - _EMBED), jnp.float32)


def _transpose_call(table_t):
    nblocks = pl.cdiv(_VOCAB, _TBLK)
    return pl.pallas_call(
        _tr_body,
        grid=(nblocks,),
        in_specs=[pl.BlockSpec((_EMBED, _TBLK), lambda j: (0, j))],
        out_specs=pl.BlockSpec((_TBLK, _EMBED_PAD), lambda j: (j, 0)),
        out_shape=jax.ShapeDtypeStruct((_VOCAB, _EMBED_PAD), jnp.float32),
    )(table_t)


def _gather_call(table_pad, idx):
    info = plsc.get_sparse_core_info()
    nc, ns = info.num_cores, info.num_subcores
    nw = nc * ns
    b_per_w = _BATCH // nw
    mesh = plsc.VectorSubcoreMesh(core_axis_name="c", subcore_axis_name="s")

    @functools.partial(
        pl.kernel,
        mesh=mesh,
        out_type=jax.ShapeDtypeStruct((_BATCH, _EMBED_PAD), jnp.float32),
        scratch_types=[
            pltpu.VMEM((b_per_w,), jnp.int32),
            pltpu.VMEM((b_per_w, _EMBED_PAD), jnp.float32),
            pltpu.SemaphoreType.DMA,
        ],
    )
    def gather_k(table_hbm, idx_hbm, out_hbm, idx_v, rows_v, sem):
        wid = lax.axis_index("s") * nc + lax.axis_index("c")
        base = wid * b_per_w
        pltpu.sync_copy(idx_hbm.at[pl.ds(base, b_per_w)], idx_v)
        pltpu.async_copy(table_hbm.at[idx_v], rows_v, sem).wait()
        pltpu.sync_copy(rows_v, out_hbm.at[pl.ds(base, b_per_w)])

    return gather_k(table_pad, idx)


def _mm_body(x_ref, w_ref, b_ref, out_ref, xs_ref):
    @pl.when(pl.program_id(0) == 0)
    def _():
        xv = x_ref[:, :_EMBED]
        ss = jnp.sum(xv * xv, axis=1, keepdims=True)
        norm = jnp.sqrt(ss)
        scale = jnp.minimum(1.0, _MAX_NORM / jnp.maximum(norm, 1e-7))
        xs_ref[...] = jnp.transpose(xv * scale)

    out_ref[...] = lax.dot_general(
        w_ref[...],
        xs_ref[...],
        dimension_numbers=(((0,), (0,)), ((), ())),
        preferred_element_type=jnp.float32,
    ) + b_ref[...][:, None]


def _matmul_call(x, W, b):
    wt = jnp.transpose(W)
    nblocks = pl.cdiv(_VOCAB, _NBLK)
    out_t = pl.pallas_call(
        _mm_body,
        grid=(nblocks,),
        in_specs=[
            pl.BlockSpec((_BATCH, _EMBED_PAD), lambda j: (0, 0)),
            pl.BlockSpec((_EMBED, _NBLK), lambda j: (0, j)),
            pl.BlockSpec((_NBLK,), lambda j: (j,)),
        ],
        out_specs=pl.BlockSpec((_NBLK, _BATCH), lambda j: (j, 0)),
        out_shape=jax.ShapeDtypeStruct((_VOCAB, _BATCH), jnp.float32),
        scratch_shapes=[
            pltpu.VMEM((_EMBED, _BATCH), jnp.float32),
        ],
    )(x, wt, b)
    return jnp.transpose(out_t)


def kernel(inputs, emb_table, W, b):
    table_pad = _transpose_call(jnp.transpose(emb_table))
    x = _gather_call(table_pad, inputs.astype(jnp.int32))
    return _matmul_call(x, W, b)
